# baseline (speedup 1.0000x reference)
import functools

import jax
import jax.numpy as jnp
from jax import lax
from jax.experimental import pallas as pl
from jax.experimental.pallas import tpu as pltpu
from jax.experimental.pallas import tpu_sc as plsc


def _matmul(h, W_T):
    N, K = h.shape
    DO = W_T.shape[1]
    RB = 1000

    def mm_body(h_ref, w_ref, o_ref):
        o_ref[...] = jnp.dot(h_ref[...], w_ref[...],
                             preferred_element_type=jnp.float32
                             ).astype(jnp.bfloat16)

    return pl.pallas_call(
        mm_body,
        grid=(N // RB,),
        in_specs=[
            pl.BlockSpec((RB, K), lambda j: (j, 0)),
            pl.BlockSpec((K, DO), lambda j: (0, 0)),
        ],
        out_specs=pl.BlockSpec((RB, DO), lambda j: (j, 0)),
        out_shape=jax.ShapeDtypeStruct((N, DO), jnp.bfloat16),
    )(h, W_T)


def _edge_aggregate(h2, row3, col3, attn3, N_PAD, DO):
    NW, NSC, SCC, K = row3.shape
    NT = 16
    R_COUNT = N_PAD // NT
    ZR = 128
    NQ = DO // 16
    NPAIR = (SCC - 3) // 2

    mesh = plsc.VectorSubcoreMesh(core_axis_name="c", subcore_axis_name="s")

    @functools.partial(
        pl.kernel,
        mesh=mesh,
        out_type=jax.ShapeDtypeStruct((2, N_PAD, DO), jnp.float32),
        scratch_types=[
            pltpu.VMEM((SCC, K), jnp.int32),
            pltpu.VMEM((SCC, K), jnp.int32),
            pltpu.VMEM((SCC, K), jnp.float32),
            pltpu.VMEM((2, K, DO // 2), jnp.int32),
            pltpu.VMEM((2, K, DO), jnp.float32),
            pltpu.VMEM_SHARED((N_PAD, DO), jnp.float32),
            pltpu.SemaphoreType.DMA,
            pltpu.SemaphoreType.DMA,
            pltpu.SemaphoreType.DMA,
            pltpu.SemaphoreType.DMA,
        ],
        compiler_params=pltpu.CompilerParams(
            needs_layout_passes=False, use_tc_tiling_on_sc=False),
    )
    def agg(h2_hbm, row_hbm, col_hbm, attn_hbm, out_hbm,
            col_b, row_b, attn_b, msg_bf, msg_v, acc_s, g0, g1, s0, s1):
        c = lax.axis_index("c")
        s = lax.axis_index("s")
        w = c * NT + s
        r_lo = s * R_COUNT
        gsem = (g0, g1)
        ssem = (s0, s1)

        zvec = jnp.zeros((16,), jnp.float32)

        def zero_body(i, carry):
            for q in range(NQ):
                msg_v[0, i, pl.ds(q * 16, 16)] = zvec
            return carry

        lax.fori_loop(0, K, zero_body, 0)
        for b in range(R_COUNT // K):
            pltpu.sync_copy(msg_v.at[0], acc_s.at[pl.ds(r_lo + b * K, K)])

        plsc.subcore_barrier()

        def start_gather(ci, b):
            pltpu.async_copy(h2_hbm.at[col_b.at[ci]], msg_bf.at[b], gsem[b])

        def wait_gather(ci, b):
            pltpu.make_async_copy(
                h2_hbm.at[col_b.at[ci]], msg_bf.at[b], gsem[b]).wait()

        def start_scatter(ci, b):
            pltpu.async_copy(msg_v.at[b], acc_s.at[row_b.at[ci]], ssem[b],
                             add=True)

        def wait_scatter(ci, b):
            pltpu.make_async_copy(
                msg_v.at[b], acc_s.at[row_b.at[ci]], ssem[b]).wait()

        def scale(ci, b):
            hi_mask = jnp.full((16,), -65536, jnp.int32)

            def group(g, carry):
                e0 = g * 16
                for l in range(16):
                    e = e0 + l
                    sp = plsc.load_gather(
                        attn_b,
                        [jnp.full((16,), 0, jnp.int32) + ci,
                         jnp.full((16,), 0, jnp.int32) + e])
                    for q in range(DO // 32):
                        w32 = msg_bf[b, e, pl.ds(q * 16, 16)]
                        lo = plsc.bitcast(w32 << 16, jnp.float32) * sp
                        hi = plsc.bitcast(w32 & hi_mask, jnp.float32) * sp
                        msg_v[b, e, pl.ds(q * 32, 16)] = lo
                        msg_v[b, e, pl.ds(q * 32 + 16, 16)] = hi
                return carry

            lax.fori_loop(0, K // 16, group, 0)

        def superchunk_body(scj, carry):
            pltpu.sync_copy(col_hbm.at[w, scj], col_b)
            pltpu.sync_copy(row_hbm.at[w, scj], row_b)
            pltpu.sync_copy(attn_hbm.at[w, scj], attn_b)

            start_gather(0, 0)
            start_gather(1, 1)

            def pair_body(j2, carry):
                c0 = 2 * j2
                wait_gather(c0, 0)
                scale(c0, 0)
                start_scatter(c0, 0)
                wait_gather(c0 + 1, 1)
                scale(c0 + 1, 1)
                start_scatter(c0 + 1, 1)
                wait_scatter(c0, 0)
                start_gather(c0 + 2, 0)
                wait_scatter(c0 + 1, 1)
                start_gather(c0 + 3, 1)
                return carry

            lax.fori_loop(0, NPAIR, pair_body, 0)

            t0 = SCC - 3
            wait_gather(t0, 0)
            scale(t0, 0)
            start_scatter(t0, 0)
            wait_gather(t0 + 1, 1)
            scale(t0 + 1, 1)
            start_scatter(t0 + 1, 1)
            wait_scatter(t0, 0)
            start_gather(t0 + 2, 0)
            wait_gather(t0 + 2, 0)
            scale(t0 + 2, 0)
            start_scatter(t0 + 2, 0)
            wait_scatter(t0 + 1, 1)
            wait_scatter(t0 + 2, 0)
            return carry

        lax.fori_loop(0, NSC, superchunk_body, 0)

        plsc.subcore_barrier()

        for b in range(R_COUNT // ZR):
            r0 = r_lo + b * ZR
            pltpu.sync_copy(acc_s.at[pl.ds(r0, ZR)],
                            out_hbm.at[c, pl.ds(r0, ZR)])

    return agg(h2, row3, col3, attn3)


def _combine(out_p):
    _, N_PAD, DO = out_p.shape
    RB = 640

    def add_body(i_ref, o_ref):
        o_ref[...] = i_ref[0] + i_ref[1]

    return pl.pallas_call(
        add_body,
        grid=(N_PAD // RB,),
        in_specs=[pl.BlockSpec((2, RB, DO), lambda j: (0, j, 0))],
        out_specs=pl.BlockSpec((RB, DO), lambda j: (j, 0)),
        out_shape=jax.ShapeDtypeStruct((N_PAD, DO), jnp.float32),
    )(out_p)


def kernel(h, edge_index, attn, W_T):
    N = h.shape[0]
    DO = W_T.shape[1]
    E = attn.shape[0]
    N_PAD = 10240
    NW, K, SCC = 32, 80, 25
    NSC = E // (NW * SCC * K)
    row3 = edge_index[0].astype(jnp.int32).reshape(NW, NSC, SCC, K)
    col3 = edge_index[1].astype(jnp.int32).reshape(NW, NSC, SCC, K)
    attn3 = attn.astype(jnp.float32).reshape(NW, NSC, SCC, K)
    g = jnp.arange(DO) // 32
    p = jnp.arange(DO) % 32
    perm = g * 32 + (p % 2) * 16 + p // 2
    W_perm = W_T.astype(jnp.float32)[:, perm]
    h2bf = _matmul(h.astype(jnp.float32), W_perm)
    h2 = jax.lax.bitcast_convert_type(
        h2bf.reshape(N, DO // 2, 2), jnp.int32)
    out_p = _edge_aggregate(h2, row3, col3, attn3, N_PAD, DO)
    return _combine(out_p)[:N]


# speedup vs baseline: 1.8371x; 1.8371x over previous
import functools

import jax
import jax.numpy as jnp
from jax import lax
from jax.experimental import pallas as pl
from jax.experimental.pallas import tpu as pltpu
from jax.experimental.pallas import tpu_sc as plsc


def _matmul(h, W_T):
    N, K = h.shape
    DO = W_T.shape[1]
    RB = 1000

    def mm_body(h_ref, w_ref, o_ref):
        o_ref[...] = jnp.dot(h_ref[...], w_ref[...],
                             preferred_element_type=jnp.float32)

    return pl.pallas_call(
        mm_body,
        grid=(N // RB,),
        in_specs=[
            pl.BlockSpec((RB, K), lambda j: (j, 0)),
            pl.BlockSpec((K, DO), lambda j: (0, 0)),
        ],
        out_specs=pl.BlockSpec((RB, DO), lambda j: (j, 0)),
        out_shape=jax.ShapeDtypeStruct((N, DO), jnp.float32),
    )(h, W_T)


def _edge_aggregate(h2, row3, col3, attn3, N_PAD, DO):
    NW, NSC, SCC, K = row3.shape
    NT = 16
    R_COUNT = N_PAD // NT
    ZR = 128
    NQ = DO // 16
    NTRI = (SCC - 4) // 3

    mesh = plsc.VectorSubcoreMesh(core_axis_name="c", subcore_axis_name="s")

    @functools.partial(
        pl.kernel,
        mesh=mesh,
        out_type=jax.ShapeDtypeStruct((2, N_PAD, DO), jnp.float32),
        scratch_types=[
            pltpu.VMEM((SCC, K), jnp.int32),
            pltpu.VMEM((SCC, K), jnp.int32),
            pltpu.VMEM((SCC, K), jnp.float32),
            pltpu.VMEM((3, K, DO), jnp.float32),
            pltpu.VMEM_SHARED((N_PAD, DO), jnp.float32),
            pltpu.SemaphoreType.DMA,
            pltpu.SemaphoreType.DMA,
            pltpu.SemaphoreType.DMA,
            pltpu.SemaphoreType.DMA,
            pltpu.SemaphoreType.DMA,
            pltpu.SemaphoreType.DMA,
        ],
        compiler_params=pltpu.CompilerParams(needs_layout_passes=False),
    )
    def agg(h2_hbm, row_hbm, col_hbm, attn_hbm, out_hbm,
            col_b, row_b, attn_b, msg_v, acc_s, g0, g1, g2, s0, s1, s2):
        c = lax.axis_index("c")
        s = lax.axis_index("s")
        w = c * NT + s
        r_lo = s * R_COUNT
        gsem = (g0, g1, g2)
        ssem = (s0, s1, s2)

        zvec = jnp.zeros((16,), jnp.float32)

        def zero_body(i, carry):
            for q in range(NQ):
                msg_v[0, i, pl.ds(q * 16, 16)] = zvec
            return carry

        lax.fori_loop(0, K, zero_body, 0)
        for b in range(R_COUNT // K):
            pltpu.sync_copy(msg_v.at[0], acc_s.at[pl.ds(r_lo + b * K, K)])

        plsc.subcore_barrier()

        def start_gather(ci, b):
            pltpu.async_copy(h2_hbm.at[col_b.at[ci]], msg_v.at[b], gsem[b])

        def wait_gather(ci, b):
            pltpu.make_async_copy(
                h2_hbm.at[col_b.at[ci]], msg_v.at[b], gsem[b]).wait()

        def start_scatter(ci, b):
            pltpu.async_copy(msg_v.at[b], acc_s.at[row_b.at[ci]], ssem[b],
                             add=True)

        def wait_scatter(ci, b):
            pltpu.make_async_copy(
                msg_v.at[b], acc_s.at[row_b.at[ci]], ssem[b]).wait()

        def scale(ci, b):
            def group(g, carry):
                e0 = g * 16
                for l in range(16):
                    e = e0 + l
                    sp = plsc.load_gather(
                        attn_b,
                        [jnp.full((16,), 0, jnp.int32) + ci,
                         jnp.full((16,), 0, jnp.int32) + e])
                    for q in range(NQ):
                        sl = pl.ds(q * 16, 16)
                        msg_v[b, e, sl] = msg_v[b, e, sl] * sp
                return carry

            lax.fori_loop(0, K // 16, group, 0)

        def superchunk_body(scj, carry):
            pltpu.sync_copy(col_hbm.at[w, scj], col_b)
            pltpu.sync_copy(row_hbm.at[w, scj], row_b)
            pltpu.sync_copy(attn_hbm.at[w, scj], attn_b)

            start_gather(0, 0)
            start_gather(1, 1)
            start_gather(2, 2)

            def tri_body(j3, carry):
                c0 = 3 * j3

                @pl.when(j3 > 0)
                def _():
                    wait_scatter(c0 - 1, 2)
                    start_gather(c0 + 2, 2)

                wait_gather(c0, 0)
                scale(c0, 0)
                start_scatter(c0, 0)
                wait_gather(c0 + 1, 1)
                scale(c0 + 1, 1)
                start_scatter(c0 + 1, 1)
                wait_scatter(c0, 0)
                start_gather(c0 + 3, 0)
                wait_gather(c0 + 2, 2)
                scale(c0 + 2, 2)
                start_scatter(c0 + 2, 2)
                wait_scatter(c0 + 1, 1)
                start_gather(c0 + 4, 1)
                return carry

            lax.fori_loop(0, NTRI, tri_body, 0)

            t0 = 3 * NTRI
            wait_scatter(t0 - 1, 2)
            start_gather(t0 + 2, 2)
            wait_gather(t0, 0)
            scale(t0, 0)
            start_scatter(t0, 0)
            wait_gather(t0 + 1, 1)
            scale(t0 + 1, 1)
            start_scatter(t0 + 1, 1)
            wait_scatter(t0, 0)
            start_gather(t0 + 3, 0)
            wait_gather(t0 + 2, 2)
            scale(t0 + 2, 2)
            start_scatter(t0 + 2, 2)
            wait_gather(t0 + 3, 0)
            scale(t0 + 3, 0)
            start_scatter(t0 + 3, 0)
            wait_scatter(t0 + 1, 1)
            wait_scatter(t0 + 2, 2)
            wait_scatter(t0 + 3, 0)
            return carry

        lax.fori_loop(0, NSC, superchunk_body, 0)

        plsc.subcore_barrier()

        for b in range(R_COUNT // ZR):
            r0 = r_lo + b * ZR
            pltpu.sync_copy(acc_s.at[pl.ds(r0, ZR)],
                            out_hbm.at[c, pl.ds(r0, ZR)])

    return agg(h2, row3, col3, attn3)


def _combine(out_p):
    _, N_PAD, DO = out_p.shape
    RB = 640

    def add_body(i_ref, o_ref):
        o_ref[...] = i_ref[0] + i_ref[1]

    return pl.pallas_call(
        add_body,
        grid=(N_PAD // RB,),
        in_specs=[pl.BlockSpec((2, RB, DO), lambda j: (0, j, 0))],
        out_specs=pl.BlockSpec((RB, DO), lambda j: (j, 0)),
        out_shape=jax.ShapeDtypeStruct((N_PAD, DO), jnp.float32),
    )(out_p)


def kernel(h, edge_index, attn, W_T):
    N = h.shape[0]
    DO = W_T.shape[1]
    E = attn.shape[0]
    N_PAD = 10240
    NW, K, SCC = 32, 80, 25
    NSC = E // (NW * SCC * K)
    row3 = edge_index[0].astype(jnp.int32).reshape(NW, NSC, SCC, K)
    col3 = edge_index[1].astype(jnp.int32).reshape(NW, NSC, SCC, K)
    attn3 = attn.astype(jnp.float32).reshape(NW, NSC, SCC, K)
    h2 = _matmul(h.astype(jnp.float32), W_T.astype(jnp.float32))
    out_p = _edge_aggregate(h2, row3, col3, attn3, N_PAD, DO)
    return _combine(out_p)[:N]
